# baseline (speedup 1.0000x reference)
import functools

import jax
import jax.numpy as jnp
import numpy as np
from jax import lax
from jax.experimental import pallas as pl
from jax.experimental.pallas import tpu as pltpu
from jax.experimental.pallas import tpu_sc as plsc

_LAG = 24
_QN = 64
_B = 32
_T = 8192
_TOUT = _T - _LAG + 1
_PADT = 8256
_NI = _T // 16 + 1
_TB = 8176

_UTAB = np.full((128,), np.inf, dtype=np.float32)
_UTAB[0] = -np.inf
_UTAB[1:65] = np.linspace(-2.0, 2.0, _QN + 1)[:-1].astype(np.float32)

_mesh = plsc.VectorSubcoreMesh(core_axis_name="c", subcore_axis_name="s")


@functools.partial(
    pl.kernel,
    out_type=jax.ShapeDtypeStruct((_B, 1, _T), jnp.int32),
    mesh=_mesh,
    scratch_types=[
        pltpu.VMEM((_T * 4,), jnp.float32),
        pltpu.VMEM((_PADT,), jnp.float32),
        pltpu.VMEM((_PADT,), jnp.float32),
        pltpu.VMEM((_PADT,), jnp.float32),
        pltpu.VMEM((_T,), jnp.int32),
    ],
    compiler_params=pltpu.CompilerParams(needs_layout_passes=False),
)
def _sc_bins(x_hbm, out_hbm, xr, xv, wa, wb, bv):
    wid = lax.axis_index("s") * 2 + lax.axis_index("c")
    iota = lax.iota(jnp.int32, 16)
    ninf = jnp.full((16,), -jnp.inf, dtype=jnp.float32)

    pltpu.sync_copy(x_hbm.at[wid], xr)

    def gbody(i, carry):
        b = i * 16
        xv[pl.ds(b, 16)] = plsc.load_gather(xr, [(b + iota) * 4 + 1])
        return carry
    lax.fori_loop(0, _T // 16, gbody, 0)

    for b in range(_T, _PADT, 16):
        xv[pl.ds(b, 16)] = ninf
        wa[pl.ds(b, 16)] = ninf
        wb[pl.ds(b, 16)] = ninf

    def mpass(dst, src, off):
        def body(i, carry):
            b = i * 16
            dst[pl.ds(b, 16)] = jnp.maximum(src[pl.ds(b, 16)],
                                            src[pl.ds(b + off, 16)])
            return carry
        lax.fori_loop(0, _NI, body, 0)

    mpass(wa, xv, 1)
    mpass(wb, wa, 2)
    mpass(wa, wb, 4)
    mpass(wb, wa, 8)

    def fbody(i, carry):
        b = i * 16
        v = jnp.maximum(wb[pl.ds(b, 16)], wa[pl.ds(b + 16, 16)])
        u = jnp.clip((v + 2.0) * 16.0, -1.0, 64.0)
        c0 = (u + 1.0).astype(jnp.int32) - 1
        blo = c0.astype(jnp.float32) * 0.0625 - 2.0
        bhi = (c0 + 1).astype(jnp.float32) * 0.0625 - 2.0
        c = c0 - (v < blo).astype(jnp.int32) + (v >= bhi).astype(jnp.int32)
        bv[pl.ds(b, 16)] = jnp.clip(c, -1, 63)
        return carry
    lax.fori_loop(0, _T // 16, fbody, 0)

    pltpu.sync_copy(bv, out_hbm.at[wid, 0])


_CH = 512
_NCHT = 16


def _tc_expand_body(bins_ref, out_ref, buf, sems):
    b = pl.program_id(0)
    row = bins_ref[0, 0, :]
    col = lax.broadcasted_iota(jnp.int32, (1, _QN), 1)
    for i in range(_NCHT):
        r0 = i * _CH
        nr = min(_CH, _TOUT - r0)
        cp = pltpu.make_async_copy(
            buf.at[i, pl.ds(0, nr)], out_ref.at[b, pl.ds(r0, nr)], sems.at[i])

        @pl.when(b > 0)
        def _wait_prev():
            pltpu.make_async_copy(
                buf.at[i, pl.ds(0, nr)],
                out_ref.at[b - 1, pl.ds(r0, nr)], sems.at[i]).wait()

        c = lax.slice(row, (r0,), (r0 + _CH,)).reshape(_CH, 1)
        buf[i] = (c == col).astype(jnp.float32)
        cp.start()

    @pl.when(b == _B - 1)
    def _drain():
        for i in range(_NCHT):
            r0 = i * _CH
            nr = min(_CH, _TOUT - r0)
            pltpu.make_async_copy(
                buf.at[i, pl.ds(0, nr)], out_ref.at[b, pl.ds(r0, nr)],
                sems.at[i]).wait()


def _tc_expand(bins):
    return pl.pallas_call(
        _tc_expand_body,
        out_shape=jax.ShapeDtypeStruct((_B, _TOUT, _QN), jnp.float32),
        grid=(_B,),
        in_specs=[pl.BlockSpec((1, 1, _T), lambda b: (b, 0, 0))],
        out_specs=pl.BlockSpec(memory_space=pltpu.MemorySpace.HBM),
        scratch_shapes=[
            pltpu.VMEM((_NCHT, _CH, _QN), jnp.float32),
            pltpu.SemaphoreType.DMA((_NCHT,)),
        ],
    )(bins)


def kernel(inp):
    bins = _sc_bins(inp.reshape(_B, _T * 4))
    return _tc_expand(bins)


# speedup vs baseline: 1.1252x; 1.1252x over previous
import functools

import jax
import jax.numpy as jnp
import numpy as np
from jax import lax
from jax.experimental import pallas as pl
from jax.experimental.pallas import tpu as pltpu
from jax.experimental.pallas import tpu_sc as plsc

_LAG = 24
_QN = 64
_B = 32
_T = 8192
_TOUT = _T - _LAG + 1
_PADT = 8320
_NI = _T // 16 + 1
_TB = 8176

_UTAB = np.full((128,), np.inf, dtype=np.float32)
_UTAB[0] = -np.inf
_UTAB[1:65] = np.linspace(-2.0, 2.0, _QN + 1)[:-1].astype(np.float32)

_mesh = plsc.VectorSubcoreMesh(core_axis_name="c", subcore_axis_name="s")


@functools.partial(
    pl.kernel,
    out_type=jax.ShapeDtypeStruct((_B, 1, _T), jnp.int32),
    mesh=_mesh,
    scratch_types=[
        pltpu.VMEM((_T * 4,), jnp.float32),
        pltpu.VMEM((_PADT,), jnp.float32),
        pltpu.VMEM((_PADT,), jnp.float32),
        pltpu.VMEM((_PADT,), jnp.float32),
        pltpu.VMEM((_T,), jnp.int32),
    ],
    compiler_params=pltpu.CompilerParams(needs_layout_passes=False),
)
def _sc_bins(x_hbm, out_hbm, xr, xv, wa, wb, bv):
    wid = lax.axis_index("s") * 2 + lax.axis_index("c")
    iota = lax.iota(jnp.int32, 16)
    ninf = jnp.full((16,), -jnp.inf, dtype=jnp.float32)

    pltpu.sync_copy(x_hbm.at[wid], xr)

    def gbody(i, carry):
        b = i * 64
        for j in range(4):
            xv[pl.ds(b + j * 16, 16)] = plsc.load_gather(
                xr, [(b + j * 16 + iota) * 4 + 1])
        return carry
    lax.fori_loop(0, _T // 64, gbody, 0)

    for b in range(_T, _PADT, 16):
        xv[pl.ds(b, 16)] = ninf
        wa[pl.ds(b, 16)] = ninf
        wb[pl.ds(b, 16)] = ninf

    def mpass(dst, src, off):
        def body(i, carry):
            b = i * 32
            dst[pl.ds(b, 16)] = jnp.maximum(src[pl.ds(b, 16)],
                                            src[pl.ds(b + off, 16)])
            dst[pl.ds(b + 16, 16)] = jnp.maximum(src[pl.ds(b + 16, 16)],
                                                 src[pl.ds(b + 16 + off, 16)])
            return carry
        lax.fori_loop(0, _NI // 2 + 1, body, 0)

    mpass(wa, xv, 1)
    mpass(wb, wa, 2)
    mpass(wa, wb, 4)
    mpass(wb, wa, 8)

    def fbody(i, carry):
        for j in range(2):
            b = i * 32 + j * 16
            v = jnp.maximum(wb[pl.ds(b, 16)], wa[pl.ds(b + 16, 16)])
            u = jnp.clip((v + 2.0) * 16.0, -1.0, 64.0)
            c0 = (u + 1.0).astype(jnp.int32) - 1
            blo = c0.astype(jnp.float32) * 0.0625 - 2.0
            bhi = (c0 + 1).astype(jnp.float32) * 0.0625 - 2.0
            c = c0 - (v < blo).astype(jnp.int32) + (v >= bhi).astype(jnp.int32)
            bv[pl.ds(b, 16)] = jnp.clip(c, -1, 63)
        return carry
    lax.fori_loop(0, _T // 32, fbody, 0)

    pltpu.sync_copy(bv, out_hbm.at[wid, 0])


def _tc_expand_body(bins_ref, out_ref):
    row = bins_ref[0, 0, :]
    c = lax.slice(row, (0,), (_TB,)).reshape(_TB, 1)
    col = lax.broadcasted_iota(jnp.int32, (1, _QN), 1)
    out_ref[0] = (c == col).astype(jnp.float32)


def _tc_expand(bins):
    return pl.pallas_call(
        _tc_expand_body,
        out_shape=jax.ShapeDtypeStruct((_B, _TOUT, _QN), jnp.float32),
        grid=(_B, -(-_TOUT // _TB)),
        in_specs=[pl.BlockSpec((1, 1, _T), lambda b, t: (b, 0, 0))],
        out_specs=pl.BlockSpec((1, _TB, _QN), lambda b, t: (b, t, 0)),
    )(bins)


def kernel(inp):
    bins = _sc_bins(inp.reshape(_B, _T * 4))
    return _tc_expand(bins)
